# Initial kernel scaffold; baseline (speedup 1.0000x reference)
import jax
import jax.numpy as jnp
from jax.experimental import pallas as pl


def kernel(x, table):
    raise NotImplementedError("write your pallas kernel here")


# speedup vs baseline: 1.4829x; 1.4829x over previous
import functools

import jax
import jax.numpy as jnp
from jax import lax
from jax.experimental import pallas as pl
from jax.experimental.pallas import tpu as pltpu
from jax.experimental.pallas import tpu_sc as plsc

VOCAB = 4096
D = 4096
B_TOK = 16384

_info = plsc.get_sparse_core_info()
NC = _info.num_cores
NS = _info.num_subcores
NW = NC * NS
BPW = B_TOK // NW
ROWS = 16
NCHUNK = BPW // ROWS


_mesh = plsc.VectorSubcoreMesh(core_axis_name="c", subcore_axis_name="s")


@functools.partial(
    pl.kernel,
    mesh=_mesh,
    out_type=jax.ShapeDtypeStruct((B_TOK, D), jnp.float32),
    scratch_types=[
        pltpu.VMEM((BPW,), jnp.int32),
        pltpu.VMEM((ROWS, D), jnp.float32),
        pltpu.SemaphoreType.DMA,
    ],
)
def _gather_rows(idx_hbm, table_hbm, out_hbm, idx_v, buf, sem):
    wid = lax.axis_index("s") * NC + lax.axis_index("c")
    base = wid * BPW
    pltpu.sync_copy(idx_hbm.at[pl.ds(base, BPW)], idx_v)

    @pl.loop(0, NCHUNK)
    def _chunk(c):
        off = c * ROWS
        pltpu.async_copy(
            table_hbm.at[idx_v.at[pl.ds(off, ROWS)]], buf, sem
        ).wait()
        pltpu.sync_copy(buf, out_hbm.at[pl.ds(base + off, ROWS)])


def kernel(x, table):
    idx = x.reshape(B_TOK).astype(jnp.int32)
    out = _gather_rows(idx, table)
    return out.reshape(x.shape[0], x.shape[1], D)


# speedup vs baseline: 1.5867x; 1.0700x over previous
import functools

import jax
import jax.numpy as jnp
from jax import lax
from jax.experimental import pallas as pl
from jax.experimental.pallas import tpu as pltpu
from jax.experimental.pallas import tpu_sc as plsc

VOCAB = 4096
D = 4096
B_TOK = 16384

_info = plsc.get_sparse_core_info()
NC = _info.num_cores
NS = _info.num_subcores
NW = NC * NS
BPW = B_TOK // NW
ROWS = 8
NCHUNK = BPW // ROWS
NBUF = 2


_mesh = plsc.VectorSubcoreMesh(core_axis_name="c", subcore_axis_name="s")


@functools.partial(
    pl.kernel,
    mesh=_mesh,
    out_type=jax.ShapeDtypeStruct((B_TOK, D), jnp.float32),
    scratch_types=[
        pltpu.VMEM((BPW,), jnp.int32),
        pltpu.VMEM((ROWS, D), jnp.float32),
        pltpu.VMEM((ROWS, D), jnp.float32),
        pltpu.SemaphoreType.DMA,
        pltpu.SemaphoreType.DMA,
        pltpu.SemaphoreType.DMA,
    ],
)
def _gather_rows(idx_hbm, table_hbm, out_hbm, idx_v, b0, b1, gsem, o0, o1):
    bufs = (b0, b1)
    osems = (o0, o1)
    wid = lax.axis_index("s") * NC + lax.axis_index("c")
    base = wid * BPW
    pltpu.sync_copy(idx_hbm.at[pl.ds(base, BPW)], idx_v)

    def gather(c, b):
        pltpu.async_copy(
            table_hbm.at[idx_v.at[pl.ds(c * ROWS, ROWS)]], bufs[b],
            gsem).wait()

    def issue_out(c, b):
        pltpu.async_copy(
            bufs[b], out_hbm.at[pl.ds(base + c * ROWS, ROWS)], osems[b])

    def drain_out(c, b):
        pltpu.make_async_copy(
            bufs[b], out_hbm.at[pl.ds(base + c * ROWS, ROWS)],
            osems[b]).wait()

    gather(0, 0)
    issue_out(0, 0)
    gather(1, 1)
    issue_out(1, 1)

    @pl.loop(0, (NCHUNK - 2) // NBUF)
    def _body(o):
        for j in range(NBUF):
            c = 2 + o * NBUF + j
            drain_out(c, j)
            gather(c, j)
            issue_out(c, j)

    drain_out(NCHUNK - 2, 0)
    drain_out(NCHUNK - 1, 1)


def kernel(x, table):
    idx = x.reshape(B_TOK).astype(jnp.int32)
    out = _gather_rows(idx, table)
    return out.reshape(x.shape[0], x.shape[1], D)


# speedup vs baseline: 1.6125x; 1.0163x over previous
import functools

import jax
import jax.numpy as jnp
from jax import lax
from jax.experimental import pallas as pl
from jax.experimental.pallas import tpu as pltpu
from jax.experimental.pallas import tpu_sc as plsc

VOCAB = 4096
D = 4096
B_TOK = 16384

_info = plsc.get_sparse_core_info()
NC = _info.num_cores
NS = _info.num_subcores
NW = NC * NS
BPW = B_TOK // NW
ROWS = 8
NCHUNK = BPW // ROWS
NBUF = 2


_mesh = plsc.VectorSubcoreMesh(core_axis_name="c", subcore_axis_name="s")


@functools.partial(
    pl.kernel,
    mesh=_mesh,
    out_type=jax.ShapeDtypeStruct((B_TOK, D), jnp.float32),
    scratch_types=[
        pltpu.VMEM((BPW,), jnp.int32),
        pltpu.VMEM((ROWS, D), jnp.float32),
        pltpu.VMEM((ROWS, D), jnp.float32),
        pltpu.SemaphoreType.DMA,
        pltpu.SemaphoreType.DMA,
        pltpu.SemaphoreType.DMA,
    ],
)
def _gather_rows(idx_hbm, table_hbm, out_hbm, idx_v, b0, b1, gsem, o0, o1):
    bufs = (b0, b1)
    osems = (o0, o1)
    wid = lax.axis_index("s") * NC + lax.axis_index("c")
    base = wid * BPW
    pltpu.sync_copy(idx_hbm.at[pl.ds(base, BPW)], idx_v)

    def issue_gather(c, b):
        return pltpu.async_copy(
            table_hbm.at[idx_v.at[pl.ds(c * ROWS, ROWS)]], bufs[b], gsem)

    def issue_out(c, b):
        pltpu.async_copy(
            bufs[b], out_hbm.at[pl.ds(base + c * ROWS, ROWS)], osems[b])

    def drain_out(c, b):
        pltpu.make_async_copy(
            bufs[b], out_hbm.at[pl.ds(base + c * ROWS, ROWS)],
            osems[b]).wait()

    def pair(c, drain):
        if drain:
            drain_out(c, 0)
        ga = issue_gather(c, 0)
        if drain:
            drain_out(c + 1, 1)
        gb = issue_gather(c + 1, 1)
        ga.wait()
        issue_out(c, 0)
        gb.wait()
        issue_out(c + 1, 1)

    pair(0, drain=False)

    @pl.loop(0, (NCHUNK - 2) // NBUF)
    def _body(o):
        pair(2 + o * NBUF, drain=True)

    drain_out(NCHUNK - 2, 0)
    drain_out(NCHUNK - 1, 1)


def kernel(x, table):
    idx = x.reshape(B_TOK).astype(jnp.int32)
    out = _gather_rows(idx, table)
    return out.reshape(x.shape[0], x.shape[1], D)
